# Initial kernel scaffold; baseline (speedup 1.0000x reference)
import jax
import jax.numpy as jnp
from jax.experimental import pallas as pl


def kernel(e, i, afe, b_edge_index, b_edge_attr, W_lin_v, b_lin_v, W_conv, b_conv, W_mlp1, b_mlp1, W_mlp2, b_mlp2, W_up, b_up, W_l0, b_l0, W_l1, b_l1, W_l2, b_l2, W_out):
    raise NotImplementedError("write your pallas kernel here")


# speedup vs baseline: 2.9228x; 2.9228x over previous
import functools

import jax
import jax.numpy as jnp
from jax import lax
from jax.experimental import pallas as pl
from jax.experimental.pallas import tpu as pltpu
from jax.experimental.pallas import tpu_sc as plsc

N = 10000
E = 320000
H = 128
AF = 7
NPAD = 10240
NC, NS = 2, 16
NW = NC * NS

_mesh = plsc.VectorSubcoreMesh(core_axis_name="c", subcore_axis_name="s")
_CP = pltpu.CompilerParams(needs_layout_passes=False)


def _tc_edge_mlp_body(attr16, w1big, b1big, w2big, b2, out):
    h = jax.nn.relu(
        lax.dot_general(attr16[...], w1big[...], (((1,), (0,)), ((), ())),
                        preferred_element_type=jnp.float32) + b1big[...])
    t = lax.dot_general(h, w2big[...], (((1,), (0,)), ((), ())),
                        preferred_element_type=jnp.float32)
    out[...] = jax.nn.sigmoid(t + b2[...])


def _tc_edge_mlp(attr16, w1big, b1big, w2big, b2):
    return pl.pallas_call(
        _tc_edge_mlp_body,
        out_shape=jax.ShapeDtypeStruct((E // 16, 16), jnp.float32),
    )(attr16, w1big, b1big, w2big, b2)


_EPT = E // NW
_NPT = NPAD // NW
_ACH = 400
_EB8 = E // 8


@functools.partial(
    pl.kernel, mesh=_mesh, compiler_params=_CP,
    out_type=(jax.ShapeDtypeStruct((NPAD, H), jnp.float32),
              jax.ShapeDtypeStruct((NW, 8, NPAD), jnp.float32)),
    scratch_types=[
        pltpu.VMEM((_NPT, H), jnp.float32),
        pltpu.VMEM((_ACH, H), jnp.float32),
        pltpu.VMEM((NPAD,), jnp.float32),
        pltpu.VMEM((_ACH,), jnp.int32),
        pltpu.VMEM((_ACH,), jnp.float32),
        pltpu.VMEM((16,), jnp.int32),
        pltpu.VMEM((_ACH + 16,), jnp.int32),
    ],
)
def _sc_agg(e_hbm, i_hbm, col_hbm, w_hbm, zn_hbm, zr_hbm, v0_hbm, degp_hbm,
            acc_v, ebuf, deg_v, cbuf, wbuf, probe_v, ibuf_v):
    cid = lax.axis_index("c")
    sid = lax.axis_index("s")
    wid = cid * NS + sid

    def _lower_bound(target):
        def _cond(st):
            lo, hi = st
            return lo < hi
        def _body(st):
            lo, hi = st
            mid = (lo + hi) // 2
            off = pl.multiple_of(mid * 8, 8)
            pltpu.sync_copy(i_hbm.at[pl.ds(off, 8)], probe_v.at[pl.ds(0, 8)])
            pv = probe_v[...]
            geq = pv[7] >= target
            return (jnp.where(geq, lo, mid + 1), jnp.where(geq, mid, hi))
        fb, _ = lax.while_loop(_cond, _body, (jnp.int32(0), jnp.int32(_EB8)))

        def _refine(fb):
            off = pl.multiple_of(fb * 8, 8)
            pltpu.sync_copy(i_hbm.at[pl.ds(off, 8)], probe_v.at[pl.ds(0, 8)])
            pv = probe_v[...]
            lt = jnp.where((pv < target) & (lax.iota(jnp.int32, 16) < 8), 1, 0)
            return fb * 8 + jnp.sum(lt).astype(jnp.int32)
        return jnp.where(fb >= _EB8, jnp.int32(E), _refine(jnp.minimum(fb, _EB8 - 1)))

    nbase = wid * _NPT
    lo = _lower_bound(nbase)
    hi = _lower_bound(nbase + _NPT)

    pltpu.sync_copy(zr_hbm, acc_v)

    lo8 = pl.multiple_of((lo // 8) * 8, 8)
    nch = (hi - lo8 + _ACH - 1) // _ACH

    def _chunk(k, done):
        off = lo8 + k * _ACH
        off = jnp.minimum(off, E - _ACH)
        off = pl.multiple_of((off // 8) * 8, 8)
        pltpu.sync_copy(e_hbm.at[1, pl.ds(off, _ACH)], ebuf)
        pltpu.sync_copy(i_hbm.at[pl.ds(off, _ACH)], ibuf_v.at[pl.ds(0, _ACH)])
        j0 = jnp.maximum(done - off, 0)
        j1 = jnp.maximum(jnp.minimum(hi - off, _ACH), j0)

        def _edge(j, _):
            jv = ibuf_v[pl.ds(j, 16)]
            r = jv[0] - nbase
            for c in range(H // 16):
                acc_v[r, pl.ds(c * 16, 16)] += ebuf[j, pl.ds(c * 16, 16)]
            return 0
        lax.fori_loop(j0, j1, _edge, 0)
        return off + j1
    lax.fori_loop(0, nch, _chunk, lo)
    pltpu.sync_copy(acc_v, v0_hbm.at[pl.ds(nbase, _NPT)])

    pltpu.sync_copy(zn_hbm, deg_v)
    ebase = wid * _EPT

    def _dchunk(k, _):
        off = ebase + k * _ACH
        pltpu.sync_copy(col_hbm.at[pl.ds(off, _ACH)], cbuf)
        pltpu.sync_copy(w_hbm.at[pl.ds(off, _ACH)], wbuf)

        def _grp(g, _):
            cv = cbuf[pl.ds(g * 16, 16)]
            wv = wbuf[pl.ds(g * 16, 16)]
            plsc.addupdate_scatter(deg_v, [cv], wv)
            return 0
        lax.fori_loop(0, _ACH // 16, _grp, 0)
        return 0
    lax.fori_loop(0, _EPT // _ACH, _dchunk, 0)
    pltpu.sync_copy(deg_v, degp_hbm.at[wid, 0])


def _tc_pre_body(v0, degp, afe, wva, wvb, blv, wconv, xsT, dis_out):
    v1 = (lax.dot_general(v0[...], wva[...], (((1,), (1,)), ((), ())),
                          preferred_element_type=jnp.float32)
          + lax.dot_general(afe[...], wvb[...], (((1,), (1,)), ((), ())),
                            preferred_element_type=jnp.float32)
          + blv[...])
    deg = jnp.sum(degp[:, 0, :], axis=0)
    dis = jnp.where(deg > 0.0, lax.rsqrt(jnp.maximum(deg, 1e-30)), 0.0)
    x1T = lax.dot_general(wconv[...], v1, (((1,), (1,)), ((), ())),
                          preferred_element_type=jnp.float32)
    xsT[...] = x1T * dis[None, :]
    dis_out[...] = dis[None, :]


def _tc_pre(v0, degp, afe, wva, wvb, blv, wconv):
    return pl.pallas_call(
        _tc_pre_body,
        out_shape=(jax.ShapeDtypeStruct((H, NPAD), jnp.float32),
                   jax.ShapeDtypeStruct((1, NPAD), jnp.float32)),
    )(v0, degp, afe, wva, wvb, blv, wconv)


_FPT = H // NW
_CCH = 2000


_HPC = H // NC
_NCH = E // _CCH


@functools.partial(
    pl.kernel, mesh=_mesh, compiler_params=_CP,
    out_type=jax.ShapeDtypeStruct((H, NPAD), jnp.float32),
    scratch_types=[
        pltpu.VMEM_SHARED((_HPC, NPAD), jnp.float32),
        pltpu.VMEM((_FPT, NPAD), jnp.float32),
        pltpu.VMEM((_FPT, NPAD), jnp.float32),
        pltpu.VMEM((_CCH,), jnp.int32),
        pltpu.VMEM((_CCH,), jnp.int32),
        pltpu.VMEM((_CCH,), jnp.float32),
    ],
)
def _sc_conv(xsT_hbm, row_hbm, col_hbm, w_hbm, zc_hbm, out_hbm,
             stage_sp, x_v, acc_v, rbuf, cbuf, wbuf):
    cid = lax.axis_index("c")
    sid = lax.axis_index("s")
    wid = cid * NS + sid

    @pl.when(sid < 8)
    def _():
        pltpu.sync_copy(xsT_hbm.at[pl.ds(cid * _HPC + sid * 8, 8)],
                        stage_sp.at[pl.ds(sid * 8, 8)])
    plsc.subcore_barrier()
    pltpu.sync_copy(stage_sp.at[pl.ds(sid * _FPT, _FPT)], x_v)
    pltpu.sync_copy(zc_hbm, acc_v)
    plsc.subcore_barrier()

    def _chunk(k, _):
        kk = k + wid * 5
        kk = jnp.where(kk >= _NCH, kk - _NCH, kk)
        off = pl.multiple_of(kk * _CCH, 8)
        pltpu.sync_copy(row_hbm.at[pl.ds(off, _CCH)], rbuf)
        pltpu.sync_copy(col_hbm.at[pl.ds(off, _CCH)], cbuf)
        pltpu.sync_copy(w_hbm.at[pl.ds(off, _CCH)], wbuf)

        def _grp(g, _):
            rv = rbuf[pl.ds(g * 16, 16)]
            cv = cbuf[pl.ds(g * 16, 16)]
            wv = wbuf[pl.ds(g * 16, 16)]
            for c in range(_FPT):
                ci = jnp.full((16,), c, jnp.int32)
                xg = plsc.load_gather(x_v, [ci, rv])
                plsc.addupdate_scatter(acc_v, [ci, cv], xg * wv)
            return 0
        lax.fori_loop(0, _CCH // 16, _grp, 0)
        return 0
    lax.fori_loop(0, _NCH, _chunk, 0)

    pltpu.sync_copy(acc_v, stage_sp.at[pl.ds(sid * _FPT, _FPT)])
    plsc.subcore_barrier()
    @pl.when(sid < 8)
    def _():
        pltpu.sync_copy(stage_sp.at[pl.ds(sid * 8, 8)],
                        out_hbm.at[pl.ds(cid * _HPC + sid * 8, 8)])


def _tc_mid_body(accT, dis, bconv, wconv, xsT):
    out1T = accT[...] * dis[...] + bconv[...]
    x2T = lax.dot_general(wconv[...], out1T, (((1,), (0,)), ((), ())),
                          preferred_element_type=jnp.float32)
    xsT[...] = x2T * dis[...]


def _tc_mid(accT, dis, bconv, wconv):
    return pl.pallas_call(
        _tc_mid_body,
        out_shape=jax.ShapeDtypeStruct((H, NPAD), jnp.float32),
    )(accT, dis, bconv, wconv)


def _tc_head_body(accT, dis, bconv, wup, bup, wl0, bl0, wl1, bl1, wl2, bl2,
                  wout, yT):
    out2T = accT[...] * dis[...] + bconv[...]
    g = jax.nn.relu(out2T)
    u = lax.dot_general(wup[...], g, (((1,), (0,)), ((), ())),
                        preferred_element_type=jnp.float32) + bup[...]
    for wref, bref in ((wl0, bl0), (wl1, bl1), (wl2, bl2)):
        u = jax.nn.relu(
            lax.dot_general(wref[...], u, (((1,), (0,)), ((), ())),
                            preferred_element_type=jnp.float32) + bref[...])
    yT[...] = lax.dot_general(wout[...], u, (((1,), (0,)), ((), ())),
                              preferred_element_type=jnp.float32)


def _tc_head(accT, dis, bconv, wup, bup, wl0, bl0, wl1, bl1, wl2, bl2, wout):
    return pl.pallas_call(
        _tc_head_body,
        out_shape=jax.ShapeDtypeStruct((1, NPAD), jnp.float32),
    )(accT, dis, bconv, wup, bup, wl0, bl0, wl1, bl1, wl2, bl2, wout)


def kernel(e, i, afe, b_edge_index, b_edge_attr, W_lin_v, b_lin_v, W_conv,
           b_conv, W_mlp1, b_mlp1, W_mlp2, b_mlp2, W_up, b_up, W_l0, b_l0,
           W_l1, b_l1, W_l2, b_l2, W_out):
    e2 = e[1]
    i32 = i.astype(jnp.int32)
    row = b_edge_index[0].astype(jnp.int32)
    col = b_edge_index[1].astype(jnp.int32)
    afe_pad = jnp.pad(afe, ((0, NPAD - N), (0, 0)))

    eye16 = jnp.eye(16, dtype=jnp.float32)
    attr16 = b_edge_attr.reshape(E // 16, 128)
    w1big = jnp.kron(eye16, W_mlp1.T)
    b1big = jnp.tile(b_mlp1, 16)[None, :]
    w2big = jnp.kron(eye16, W_mlp2.T)
    w = _tc_edge_mlp(attr16, w1big, b1big, w2big, b_mlp2.reshape(1, 1))
    w_flat = w.reshape(E)

    zn = jnp.zeros((NPAD,), jnp.float32)
    zr = jnp.zeros((_NPT, H), jnp.float32)
    zc = jnp.zeros((_FPT, NPAD), jnp.float32)

    v0, degp = _sc_agg(e, i32, col, w_flat, zn, zr)

    xs1T, dis = _tc_pre(v0, degp, afe_pad,
                        W_lin_v[:, :H], W_lin_v[:, H:], b_lin_v[None, :],
                        W_conv)

    acc1T = _sc_conv(xs1T, row, col, w_flat, zc)
    xs2T = _tc_mid(acc1T, dis, b_conv.reshape(H, 1), W_conv)
    acc2T = _sc_conv(xs2T, row, col, w_flat, zc)

    yT = _tc_head(acc2T, dis, b_conv.reshape(H, 1),
                  W_up, b_up.reshape(-1, 1), W_l0, b_l0.reshape(-1, 1),
                  W_l1, b_l1.reshape(-1, 1), W_l2, b_l2.reshape(-1, 1),
                  W_out)
    return yT.reshape(NPAD, 1)[:N]


# speedup vs baseline: 2.9798x; 1.0195x over previous
import functools

import jax
import jax.numpy as jnp
from jax import lax
from jax.experimental import pallas as pl
from jax.experimental.pallas import tpu as pltpu
from jax.experimental.pallas import tpu_sc as plsc

N = 10000
E = 320000
H = 128
AF = 7
NPAD = 10240
NC, NS = 2, 16
NW = NC * NS

_mesh = plsc.VectorSubcoreMesh(core_axis_name="c", subcore_axis_name="s")
_CP = pltpu.CompilerParams(needs_layout_passes=False)


def _tc_edge_mlp_body(attr16, w1big, b1big, w2big, b2, out):
    h = jax.nn.relu(
        lax.dot_general(attr16[...], w1big[...], (((1,), (0,)), ((), ())),
                        preferred_element_type=jnp.float32) + b1big[...])
    t = lax.dot_general(h, w2big[...], (((1,), (0,)), ((), ())),
                        preferred_element_type=jnp.float32)
    out[...] = jax.nn.sigmoid(t + b2[...])


def _tc_edge_mlp(attr16, w1big, b1big, w2big, b2):
    return pl.pallas_call(
        _tc_edge_mlp_body,
        out_shape=jax.ShapeDtypeStruct((E // 16, 16), jnp.float32),
    )(attr16, w1big, b1big, w2big, b2)


_EPT = E // NW
_NPT = NPAD // NW
_ACH = 400
_EB8 = E // 8


@functools.partial(
    pl.kernel, mesh=_mesh, compiler_params=_CP,
    out_type=(jax.ShapeDtypeStruct((NPAD, H), jnp.float32),
              jax.ShapeDtypeStruct((NW, 8, NPAD), jnp.float32)),
    scratch_types=[
        pltpu.VMEM((_NPT, H), jnp.float32),
        pltpu.VMEM((_ACH, H), jnp.float32),
        pltpu.VMEM((NPAD,), jnp.float32),
        pltpu.VMEM((_ACH,), jnp.int32),
        pltpu.VMEM((_ACH,), jnp.float32),
        pltpu.VMEM((16,), jnp.int32),
        pltpu.VMEM((_ACH + 16,), jnp.int32),
    ],
)
def _sc_agg(e_hbm, i_hbm, col_hbm, w_hbm, zn_hbm, zr_hbm, v0_hbm, degp_hbm,
            acc_v, ebuf, deg_v, cbuf, wbuf, probe_v, ibuf_v):
    cid = lax.axis_index("c")
    sid = lax.axis_index("s")
    wid = cid * NS + sid

    def _lower_bound(target):
        def _cond(st):
            lo, hi = st
            return lo < hi
        def _body(st):
            lo, hi = st
            mid = (lo + hi) // 2
            off = pl.multiple_of(mid * 8, 8)
            pltpu.sync_copy(i_hbm.at[pl.ds(off, 8)], probe_v.at[pl.ds(0, 8)])
            pv = probe_v[...]
            geq = pv[7] >= target
            return (jnp.where(geq, lo, mid + 1), jnp.where(geq, mid, hi))
        fb, _ = lax.while_loop(_cond, _body, (jnp.int32(0), jnp.int32(_EB8)))

        def _refine(fb):
            off = pl.multiple_of(fb * 8, 8)
            pltpu.sync_copy(i_hbm.at[pl.ds(off, 8)], probe_v.at[pl.ds(0, 8)])
            pv = probe_v[...]
            lt = jnp.where((pv < target) & (lax.iota(jnp.int32, 16) < 8), 1, 0)
            return fb * 8 + jnp.sum(lt).astype(jnp.int32)
        return jnp.where(fb >= _EB8, jnp.int32(E), _refine(jnp.minimum(fb, _EB8 - 1)))

    nbase = wid * _NPT
    lo = _lower_bound(nbase)
    hi = _lower_bound(nbase + _NPT)

    pltpu.sync_copy(zr_hbm, acc_v)

    lo8 = pl.multiple_of((lo // 8) * 8, 8)
    nch = (hi - lo8 + _ACH - 1) // _ACH

    def _chunk(k, done):
        off = lo8 + k * _ACH
        off = jnp.minimum(off, E - _ACH)
        off = pl.multiple_of((off // 8) * 8, 8)
        pltpu.sync_copy(e_hbm.at[1, pl.ds(off, _ACH)], ebuf)
        pltpu.sync_copy(i_hbm.at[pl.ds(off, _ACH)], ibuf_v.at[pl.ds(0, _ACH)])
        j0 = jnp.maximum(done - off, 0)
        j1 = jnp.maximum(jnp.minimum(hi - off, _ACH), j0)

        iota16 = lax.iota(jnp.int32, 16)

        def _edge(j, _):
            ib = plsc.load_gather(ibuf_v, [jnp.full((16,), j, jnp.int32)])
            ib0 = ib - nbase
            for c in range(H // 16):
                ev = ebuf[j, pl.ds(c * 16, 16)]
                plsc.addupdate_scatter(acc_v, [ib0, iota16 + (c * 16)], ev)
            return 0
        lax.fori_loop(j0, j1, _edge, 0)
        return off + j1
    lax.fori_loop(0, nch, _chunk, lo)
    pltpu.sync_copy(acc_v, v0_hbm.at[pl.ds(nbase, _NPT)])

    pltpu.sync_copy(zn_hbm, deg_v)
    ebase = wid * _EPT

    def _dchunk(k, _):
        off = ebase + k * _ACH
        pltpu.sync_copy(col_hbm.at[pl.ds(off, _ACH)], cbuf)
        pltpu.sync_copy(w_hbm.at[pl.ds(off, _ACH)], wbuf)

        def _grp(g, _):
            cv = cbuf[pl.ds(g * 16, 16)]
            wv = wbuf[pl.ds(g * 16, 16)]
            plsc.addupdate_scatter(deg_v, [cv], wv)
            return 0
        lax.fori_loop(0, _ACH // 16, _grp, 0)
        return 0
    lax.fori_loop(0, _EPT // _ACH, _dchunk, 0)
    pltpu.sync_copy(deg_v, degp_hbm.at[wid, 0])


def _tc_pre_body(v0, degp, afe, wva, wvb, blv, wconv, xsT, dis_out):
    v1 = (lax.dot_general(v0[...], wva[...], (((1,), (1,)), ((), ())),
                          preferred_element_type=jnp.float32)
          + lax.dot_general(afe[...], wvb[...], (((1,), (1,)), ((), ())),
                            preferred_element_type=jnp.float32)
          + blv[...])
    deg = jnp.sum(degp[:, 0, :], axis=0)
    dis = jnp.where(deg > 0.0, lax.rsqrt(jnp.maximum(deg, 1e-30)), 0.0)
    x1T = lax.dot_general(wconv[...], v1, (((1,), (1,)), ((), ())),
                          preferred_element_type=jnp.float32)
    xsT[...] = x1T * dis[None, :]
    dis_out[...] = dis[None, :]


def _tc_pre(v0, degp, afe, wva, wvb, blv, wconv):
    return pl.pallas_call(
        _tc_pre_body,
        out_shape=(jax.ShapeDtypeStruct((H, NPAD), jnp.float32),
                   jax.ShapeDtypeStruct((1, NPAD), jnp.float32)),
    )(v0, degp, afe, wva, wvb, blv, wconv)


_FPT = H // NW
_CCH = 1600
_UNR = 4


_HPC = H // NC
_NCH = E // _CCH


@functools.partial(
    pl.kernel, mesh=_mesh, compiler_params=_CP,
    out_type=jax.ShapeDtypeStruct((H, NPAD), jnp.float32),
    scratch_types=[
        pltpu.VMEM_SHARED((_HPC, NPAD), jnp.float32),
        pltpu.VMEM((_FPT, NPAD), jnp.float32),
        pltpu.VMEM((_FPT, NPAD), jnp.float32),
        pltpu.VMEM((_CCH,), jnp.int32),
        pltpu.VMEM((_CCH,), jnp.int32),
        pltpu.VMEM((_CCH,), jnp.float32),
    ],
)
def _sc_conv(xsT_hbm, row_hbm, col_hbm, w_hbm, zc_hbm, out_hbm,
             stage_sp, x_v, acc_v, rbuf, cbuf, wbuf):
    cid = lax.axis_index("c")
    sid = lax.axis_index("s")
    wid = cid * NS + sid

    @pl.when(sid < 8)
    def _():
        pltpu.sync_copy(xsT_hbm.at[pl.ds(cid * _HPC + sid * 8, 8)],
                        stage_sp.at[pl.ds(sid * 8, 8)])
    plsc.subcore_barrier()
    pltpu.sync_copy(stage_sp.at[pl.ds(sid * _FPT, _FPT)], x_v)
    pltpu.sync_copy(zc_hbm, acc_v)
    plsc.subcore_barrier()

    def _chunk(k, _):
        kk = k + wid * 5
        kk = jnp.where(kk >= _NCH, kk - _NCH, kk)
        off = pl.multiple_of(kk * _CCH, 8)
        pltpu.sync_copy(row_hbm.at[pl.ds(off, _CCH)], rbuf)
        pltpu.sync_copy(col_hbm.at[pl.ds(off, _CCH)], cbuf)
        pltpu.sync_copy(w_hbm.at[pl.ds(off, _CCH)], wbuf)

        def _grp(g, _):
            for u in range(_UNR):
                b = g * (16 * _UNR) + u * 16
                rv = rbuf[pl.ds(b, 16)]
                cv = cbuf[pl.ds(b, 16)]
                wv = wbuf[pl.ds(b, 16)]
                for c in range(_FPT):
                    ci = jnp.full((16,), c, jnp.int32)
                    xg = plsc.load_gather(x_v, [ci, rv])
                    plsc.addupdate_scatter(acc_v, [ci, cv], xg * wv)
            return 0
        lax.fori_loop(0, _CCH // (16 * _UNR), _grp, 0)
        return 0
    lax.fori_loop(0, _NCH, _chunk, 0)

    pltpu.sync_copy(acc_v, stage_sp.at[pl.ds(sid * _FPT, _FPT)])
    plsc.subcore_barrier()
    @pl.when(sid < 8)
    def _():
        pltpu.sync_copy(stage_sp.at[pl.ds(sid * 8, 8)],
                        out_hbm.at[pl.ds(cid * _HPC + sid * 8, 8)])


def _tc_mid_body(accT, dis, bconv, wconv, xsT):
    out1T = accT[...] * dis[...] + bconv[...]
    x2T = lax.dot_general(wconv[...], out1T, (((1,), (0,)), ((), ())),
                          preferred_element_type=jnp.float32)
    xsT[...] = x2T * dis[...]


def _tc_mid(accT, dis, bconv, wconv):
    return pl.pallas_call(
        _tc_mid_body,
        out_shape=jax.ShapeDtypeStruct((H, NPAD), jnp.float32),
    )(accT, dis, bconv, wconv)


def _tc_head_body(accT, dis, bconv, wup, bup, wl0, bl0, wl1, bl1, wl2, bl2,
                  wout, yT):
    out2T = accT[...] * dis[...] + bconv[...]
    g = jax.nn.relu(out2T)
    u = lax.dot_general(wup[...], g, (((1,), (0,)), ((), ())),
                        preferred_element_type=jnp.float32) + bup[...]
    for wref, bref in ((wl0, bl0), (wl1, bl1), (wl2, bl2)):
        u = jax.nn.relu(
            lax.dot_general(wref[...], u, (((1,), (0,)), ((), ())),
                            preferred_element_type=jnp.float32) + bref[...])
    yT[...] = lax.dot_general(wout[...], u, (((1,), (0,)), ((), ())),
                              preferred_element_type=jnp.float32)


def _tc_head(accT, dis, bconv, wup, bup, wl0, bl0, wl1, bl1, wl2, bl2, wout):
    return pl.pallas_call(
        _tc_head_body,
        out_shape=jax.ShapeDtypeStruct((1, NPAD), jnp.float32),
    )(accT, dis, bconv, wup, bup, wl0, bl0, wl1, bl1, wl2, bl2, wout)


def kernel(e, i, afe, b_edge_index, b_edge_attr, W_lin_v, b_lin_v, W_conv,
           b_conv, W_mlp1, b_mlp1, W_mlp2, b_mlp2, W_up, b_up, W_l0, b_l0,
           W_l1, b_l1, W_l2, b_l2, W_out):
    e2 = e[1]
    i32 = i.astype(jnp.int32)
    row = b_edge_index[0].astype(jnp.int32)
    col = b_edge_index[1].astype(jnp.int32)
    afe_pad = jnp.pad(afe, ((0, NPAD - N), (0, 0)))

    eye16 = jnp.eye(16, dtype=jnp.float32)
    attr16 = b_edge_attr.reshape(E // 16, 128)
    w1big = jnp.kron(eye16, W_mlp1.T)
    b1big = jnp.tile(b_mlp1, 16)[None, :]
    w2big = jnp.kron(eye16, W_mlp2.T)
    w = _tc_edge_mlp(attr16, w1big, b1big, w2big, b_mlp2.reshape(1, 1))
    w_flat = w.reshape(E)

    zn = jnp.zeros((NPAD,), jnp.float32)
    zr = jnp.zeros((_NPT, H), jnp.float32)
    zc = jnp.zeros((_FPT, NPAD), jnp.float32)

    v0, degp = _sc_agg(e, i32, col, w_flat, zn, zr)

    xs1T, dis = _tc_pre(v0, degp, afe_pad,
                        W_lin_v[:, :H], W_lin_v[:, H:], b_lin_v[None, :],
                        W_conv)

    acc1T = _sc_conv(xs1T, row, col, w_flat, zc)
    xs2T = _tc_mid(acc1T, dis, b_conv.reshape(H, 1), W_conv)
    acc2T = _sc_conv(xs2T, row, col, w_flat, zc)

    yT = _tc_head(acc2T, dis, b_conv.reshape(H, 1),
                  W_up, b_up.reshape(-1, 1), W_l0, b_l0.reshape(-1, 1),
                  W_l1, b_l1.reshape(-1, 1), W_l2, b_l2.reshape(-1, 1),
                  W_out)
    return yT.reshape(NPAD, 1)[:N]
